# baseline (speedup 1.0000x reference)
import functools

import jax
import jax.numpy as jnp
from jax import lax
from jax.experimental import pallas as pl
from jax.experimental.pallas import tpu as pltpu
from jax.experimental.pallas import tpu_sc as plsc

_B, _C_IN, _T = 8, 96, 1024
_C_OUT, _K = 32, 512
_BB = 2
_TT = _BB * _T
_N = _B * _T


def _vq_body(x_ref, w_ref, b_ref, cb_ref, idx_ref, norms_ref):
    zs = [
        lax.dot_general(
            x_ref[i], w_ref[...], (((0,), (1,)), ((), ())),
            preferred_element_type=jnp.float32)
        for i in range(_BB)
    ]
    z = jnp.concatenate(zs, axis=0) + b_ref[...]

    zz = jnp.sum(z * z, axis=1, keepdims=True)
    cb = cb_ref[...]
    cn = jnp.sum(cb * cb, axis=1)
    s2 = lax.dot_general(
        z, cb + cb, (((1,), (1,)), ((), ())),
        preferred_element_type=jnp.float32)

    d2 = (zz - s2) + cn[None, :]

    m = jnp.min(d2, axis=1, keepdims=True)
    kio = lax.broadcasted_iota(jnp.int32, (_TT, _K), 1)
    idx = jnp.min(jnp.where(d2 == m, kio, _K), axis=1, keepdims=True)

    idx_ref[...] = idx
    for i in range(_BB):
        norms_ref[i] = jnp.broadcast_to(m[i * _T:(i + 1) * _T], (_T, 2))


def _tc_stage(x, W, b_col, codebook):
    grid = (_B // _BB,)
    return pl.pallas_call(
        _vq_body,
        grid=grid,
        in_specs=[
            pl.BlockSpec((_BB, _C_IN, _T), lambda i: (i, 0, 0)),
            pl.BlockSpec((_C_OUT, _C_IN), lambda i: (0, 0)),
            pl.BlockSpec((1, _C_OUT), lambda i: (0, 0)),
            pl.BlockSpec((_K, _C_OUT), lambda i: (0, 0)),
        ],
        out_specs=[
            pl.BlockSpec((_TT, 1), lambda i: (i, 0)),
            pl.BlockSpec((_BB, _T, 2), lambda i: (i, 0, 0)),
        ],
        out_shape=[
            jax.ShapeDtypeStruct((_N, 1), jnp.int32),
            jax.ShapeDtypeStruct((_B, _T, 2), jnp.float32),
        ],
        compiler_params=pltpu.CompilerParams(
            dimension_semantics=("parallel",)),
    )(x, W, b_col, codebook)


def _sc_gather(codebook, idx_flat):
    info = plsc.get_sparse_core_info()
    nw = info.num_cores * info.num_subcores
    bpw = _N // nw
    mesh = plsc.VectorSubcoreMesh(core_axis_name="c", subcore_axis_name="s")

    @functools.partial(
        pl.kernel, mesh=mesh,
        out_type=jax.ShapeDtypeStruct((_N, 128), jnp.float32),
        scratch_types=[
            pltpu.VMEM((bpw,), jnp.int32),
            pltpu.VMEM((bpw, 128), jnp.float32),
            pltpu.SemaphoreType.DMA,
        ],
    )
    def gather_k(table_hbm, idx_hbm, out_hbm, idx_v, rows_v, sem):
        wid = lax.axis_index("s") * info.num_cores + lax.axis_index("c")
        base = wid * bpw
        pltpu.sync_copy(idx_hbm.at[pl.ds(base, bpw)], idx_v)
        pltpu.async_copy(table_hbm.at[idx_v], rows_v, sem).wait()
        pltpu.sync_copy(rows_v, out_hbm.at[pl.ds(base, bpw)])

    return gather_k(codebook, idx_flat)


def kernel(x, W, b, codebook):
    b_col = b[None, :]
    cb_pad = jnp.pad(codebook, ((0, 0), (0, 128 - _C_OUT)))
    idx, norms = _tc_stage(x, W, b_col, codebook)
    q = _sc_gather(cb_pad, idx.reshape(_N))
    quant = jnp.transpose(q[:, :_C_OUT].reshape(_B, _T, _C_OUT), (0, 2, 1))
    return quant, norms


# speedup vs baseline: 2.4150x; 2.4150x over previous
import jax
import jax.numpy as jnp
from jax import lax
from jax.experimental import pallas as pl
from jax.experimental.pallas import tpu as pltpu

_B, _C_IN, _T = 8, 96, 1024
_C_OUT, _K = 32, 512
_BB = 2
_TT = _BB * _T


def _vq_body(x_ref, w_ref, b_ref, cb_ref, quant_ref, norms_ref):
    zs = [
        lax.dot_general(
            x_ref[i], w_ref[...], (((0,), (1,)), ((), ())),
            preferred_element_type=jnp.float32)
        for i in range(_BB)
    ]
    z = jnp.concatenate(zs, axis=0) + b_ref[...]

    zz = jnp.sum(z * z, axis=1, keepdims=True)
    cb = cb_ref[...]
    cn = jnp.sum(cb * cb, axis=1)
    s2 = lax.dot_general(
        z, cb + cb, (((1,), (1,)), ((), ())),
        preferred_element_type=jnp.float32)

    d2 = (zz - s2) + cn[None, :]

    m = jnp.min(d2, axis=1, keepdims=True)
    oh = (d2 == m).astype(jnp.bfloat16)
    count = jnp.sum(oh, axis=1, keepdims=True)

    def _tie_break():
        kio = lax.broadcasted_iota(jnp.int32, (_TT, _K), 1)
        idx = jnp.min(jnp.where(d2 == m, kio, _K), axis=1, keepdims=True)
        return (kio == idx).astype(jnp.bfloat16)

    onehot = lax.cond(
        jnp.max(count.astype(jnp.float32)) > 1.5, _tie_break, lambda: oh)

    qT = lax.dot_general(
        cb.astype(jnp.bfloat16), onehot, (((0,), (1,)), ((), ())),
        preferred_element_type=jnp.float32)

    for i in range(_BB):
        quant_ref[i] = qT[:, i * _T:(i + 1) * _T]
        norms_ref[i] = jnp.broadcast_to(m[i * _T:(i + 1) * _T], (_T, 2))


def kernel(x, W, b, codebook):
    b_col = b[None, :]
    grid = (_B // _BB,)
    quant, norms = pl.pallas_call(
        _vq_body,
        grid=grid,
        in_specs=[
            pl.BlockSpec((_BB, _C_IN, _T), lambda i: (i, 0, 0)),
            pl.BlockSpec((_C_OUT, _C_IN), lambda i: (0, 0)),
            pl.BlockSpec((1, _C_OUT), lambda i: (0, 0)),
            pl.BlockSpec((_K, _C_OUT), lambda i: (0, 0)),
        ],
        out_specs=[
            pl.BlockSpec((_BB, _C_OUT, _T), lambda i: (i, 0, 0)),
            pl.BlockSpec((_BB, _T, 2), lambda i: (i, 0, 0)),
        ],
        out_shape=[
            jax.ShapeDtypeStruct((_B, _C_OUT, _T), jnp.float32),
            jax.ShapeDtypeStruct((_B, _T, 2), jnp.float32),
        ],
        compiler_params=pltpu.CompilerParams(
            dimension_semantics=("parallel",)),
    )(x, W, b_col, codebook)
    return quant, norms


# speedup vs baseline: 2.7568x; 1.1415x over previous
import jax
import jax.numpy as jnp
from jax import lax
from jax.experimental import pallas as pl
from jax.experimental.pallas import tpu as pltpu

_B, _C_IN, _T = 8, 96, 1024
_C_OUT, _K = 32, 512
_BB = 2
_TT = _BB * _T


def _vq_body(x_ref, w_ref, b_ref, cb_ref, quant_ref, norms_ref):
    zs = [
        lax.dot_general(
            x_ref[i], w_ref[...], (((0,), (1,)), ((), ())),
            preferred_element_type=jnp.float32)
        for i in range(_BB)
    ]
    z = jnp.concatenate(zs, axis=0) + b_ref[...]

    zz = jnp.sum(z * z, axis=1, keepdims=True)
    cb = cb_ref[...]
    cn = jnp.sum(cb * cb, axis=1)
    s2 = lax.dot_general(
        z, cb + cb, (((1,), (1,)), ((), ())),
        preferred_element_type=jnp.float32)

    d2 = (zz - s2) + cn[None, :]

    m = jnp.min(d2, axis=1, keepdims=True)
    kio = lax.broadcasted_iota(jnp.int32, (_TT, _K), 1)
    idx = jnp.min(jnp.where(d2 == m, kio, _K), axis=1, keepdims=True)
    onehot = (kio == idx).astype(jnp.bfloat16)

    qT = lax.dot_general(
        cb.astype(jnp.bfloat16), onehot, (((0,), (1,)), ((), ())),
        preferred_element_type=jnp.float32)

    for i in range(_BB):
        quant_ref[i] = qT[:, i * _T:(i + 1) * _T]
        norms_ref[i] = jnp.broadcast_to(m[i * _T:(i + 1) * _T], (_T, 2))


def kernel(x, W, b, codebook):
    b_col = b[None, :]
    grid = (_B // _BB,)
    quant, norms = pl.pallas_call(
        _vq_body,
        grid=grid,
        in_specs=[
            pl.BlockSpec((_BB, _C_IN, _T), lambda i: (i, 0, 0)),
            pl.BlockSpec((_C_OUT, _C_IN), lambda i: (0, 0)),
            pl.BlockSpec((1, _C_OUT), lambda i: (0, 0)),
            pl.BlockSpec((_K, _C_OUT), lambda i: (0, 0)),
        ],
        out_specs=[
            pl.BlockSpec((_BB, _C_OUT, _T), lambda i: (i, 0, 0)),
            pl.BlockSpec((_BB, _T, 2), lambda i: (i, 0, 0)),
        ],
        out_shape=[
            jax.ShapeDtypeStruct((_B, _C_OUT, _T), jnp.float32),
            jax.ShapeDtypeStruct((_B, _T, 2), jnp.float32),
        ],
        compiler_params=pltpu.CompilerParams(
            dimension_semantics=("parallel",)),
    )(x, W, b_col, codebook)
    return quant, norms
